# baseline (speedup 1.0000x reference)
import functools

import jax
import jax.numpy as jnp
from jax import lax
from jax.experimental import pallas as pl
from jax.experimental.pallas import tpu as pltpu
from jax.experimental.pallas import tpu_sc as plsc

NC, NS, L = 2, 16, 16
NW = NC * NS
B, S, V, D = 4, 2048, 100000, 768
N = B * S
TPW = N // NW
C = 16
NCH = TPW // C
PR = C // B
NV = D // L
HR = 4
NBI = 4
NBO = 2


def _body(idx_hbm, oidx_hbm, w_hbm, segd_hbm, pe_hbm, tok_hbm,
          out_hbm, idx_v, oidx_v, w_v, segd_v, pe_v, tok_v, res_v,
          tok_sem, pe_sem, w_sem, out_sem, misc_sem):
    wid = lax.axis_index("s") * NC + lax.axis_index("c")

    cd_idx = pltpu.async_copy(idx_hbm.at[wid], idx_v, misc_sem)
    cd_oidx = pltpu.async_copy(oidx_hbm.at[wid], oidx_v, misc_sem)
    cd_segd = pltpu.async_copy(segd_hbm, segd_v, misc_sem)
    cd_idx.wait()

    def start_in(g):
        slot = lax.rem(g, NBI)
        pltpu.async_copy(
            tok_hbm.at[idx_v.at[g]], tok_v.at[slot], tok_sem.at[slot])
        pltpu.async_copy(
            pe_hbm.at[wid, g], pe_v.at[slot], pe_sem.at[slot])
        pltpu.async_copy(
            w_hbm.at[wid, g], w_v.at[slot], w_sem.at[slot])

    def wait_in(g, slot):
        pltpu.make_async_copy(
            tok_hbm.at[idx_v.at[g]], tok_v.at[slot], tok_sem.at[slot]).wait()
        pltpu.make_async_copy(
            pe_hbm.at[wid, g], pe_v.at[slot], pe_sem.at[slot]).wait()
        pltpu.make_async_copy(
            w_hbm.at[wid, g], w_v.at[slot], w_sem.at[slot]).wait()

    def start_out(g, oslot):
        pltpu.async_copy(
            res_v.at[oslot], out_hbm.at[oidx_v.at[g]], out_sem.at[oslot])

    def wait_out(g, oslot):
        pltpu.make_async_copy(
            res_v.at[oslot], out_hbm.at[oidx_v.at[g]],
            out_sem.at[oslot]).wait()

    def compute(slot, oslot):
        for h in range(C // HR):
            was = tuple(w_v[slot, h * HR + i, 0, :] for i in range(HR))
            wbs = tuple(w_v[slot, h * HR + i, 1, :] for i in range(HR))

            def jbody(j, carry):
                was_, wbs_ = carry
                off = j * L
                a1 = segd_v[pl.ds(off, L)]
                a2 = segd_v[pl.ds(D + off, L)]
                for i in range(HR):
                    row = h * HR + i
                    t = tok_v[slot, row, pl.ds(off, L)]
                    p = pe_v[slot, row // B, pl.ds(off, L)]
                    res_v[oslot, row, pl.ds(off, L)] = (
                        t + p + was_[i] * a1 + wbs_[i] * a2)
                return was_, wbs_

            plsc.parallel_loop(0, NV, 1, unroll=2, carry=(was, wbs))(jbody)

    for g0 in range(NBI):
        start_in(g0)
    cd_oidx.wait()
    cd_segd.wait()

    def gbody(g, carry):
        slot = lax.rem(g, NBI)
        oslot = lax.rem(g, NBO)
        wait_in(g, slot)

        @pl.when(g >= NBO)
        def _():
            wait_out(g - NBO, oslot)

        compute(slot, oslot)
        start_out(g, oslot)

        @pl.when(g + NBI < NCH)
        def _():
            start_in(g + NBI)

        return carry

    lax.fori_loop(0, NCH, gbody, 0)
    wait_out(NCH - 2, lax.rem(NCH - 2, NBO))
    wait_out(NCH - 1, lax.rem(NCH - 1, NBO))


_sc_call = functools.partial(
    pl.kernel,
    out_type=jax.ShapeDtypeStruct((N, D), jnp.float32),
    mesh=plsc.VectorSubcoreMesh(core_axis_name="c", subcore_axis_name="s"),
    scratch_types=[
        pltpu.VMEM((NCH, C), jnp.int32),
        pltpu.VMEM((NCH, C), jnp.int32),
        pltpu.VMEM((NBI, C, 2, L), jnp.float32),
        pltpu.VMEM((2 * D,), jnp.float32),
        pltpu.VMEM((NBI, PR, D), jnp.float32),
        pltpu.VMEM((NBI, C, D), jnp.float32),
        pltpu.VMEM((NBO, C, D), jnp.float32),
        pltpu.SemaphoreType.DMA((NBI,)),
        pltpu.SemaphoreType.DMA((NBI,)),
        pltpu.SemaphoreType.DMA((NBI,)),
        pltpu.SemaphoreType.DMA((NBO,)),
        pltpu.SemaphoreType.DMA,
    ],
)(_body)


def kernel(sequence, segment_labels, tok_table, seg_table, pe):
    seq_sm = sequence.T.reshape(NW, NCH, C).astype(jnp.int32)
    lbl_sm = segment_labels.T.reshape(NW, TPW).astype(jnp.int32)
    w = jnp.broadcast_to(
        jnp.stack([(lbl_sm >= 1), (lbl_sm >= 2)], axis=-1)
        .astype(jnp.float32)[..., None],
        (NW, TPW, 2, L)).reshape(NW, NCH, C, 2, L)
    tp = jnp.arange(N, dtype=jnp.int32)
    oidx = ((tp % B) * S + tp // B).reshape(NW, NCH, C)
    segd = jnp.concatenate(
        [seg_table[1] - seg_table[0], seg_table[2] - seg_table[1]])
    pe5 = (pe.reshape(S, D) + seg_table[0]).reshape(NW, NCH, PR, D)
    out = _sc_call(seq_sm, oidx, w, segd, pe5, tok_table)
    return out.reshape(B, S, D)


# speedup vs baseline: 1.0606x; 1.0606x over previous
import functools

import jax
import jax.numpy as jnp
from jax import lax
from jax.experimental import pallas as pl
from jax.experimental.pallas import tpu as pltpu
from jax.experimental.pallas import tpu_sc as plsc

NC, NS, L = 2, 16, 16
NW = NC * NS
B, S, V, D = 4, 2048, 100000, 768
N = B * S
TPW = N // NW
C = 32
NCH = TPW // C
PR = C // B
NV = D // L
HR = 4
SPW = TPW // B


def _body(idx_hbm, w_hbm, segd_hbm, pe_hbm, tok_hbm,
          out_hbm, idx_v, w_v, segd_v, pe_v, tok_v, res_v,
          tok_sem, pe_sem, w_sem, out_sem, misc_sem):
    wid = lax.axis_index("s") * NC + lax.axis_index("c")
    s0w = wid * SPW

    cd_idx = pltpu.async_copy(idx_hbm.at[wid], idx_v, misc_sem)
    cd_segd = pltpu.async_copy(segd_hbm, segd_v, misc_sem)
    cd_idx.wait()

    def start_in(g):
        slot = lax.rem(g, 2)
        pltpu.async_copy(
            tok_hbm.at[idx_v.at[g]], tok_v.at[slot], tok_sem.at[slot])
        pltpu.async_copy(
            pe_hbm.at[wid, g], pe_v.at[slot], pe_sem.at[slot])
        pltpu.async_copy(
            w_hbm.at[wid, g], w_v.at[slot], w_sem.at[slot])

    def wait_in(g, slot):
        pltpu.make_async_copy(
            tok_hbm.at[idx_v.at[g]], tok_v.at[slot], tok_sem.at[slot]).wait()
        pltpu.make_async_copy(
            pe_hbm.at[wid, g], pe_v.at[slot], pe_sem.at[slot]).wait()
        pltpu.make_async_copy(
            w_hbm.at[wid, g], w_v.at[slot], w_sem.at[slot]).wait()

    def start_out(g, slot):
        s0 = s0w + g * PR
        for b in range(B):
            pltpu.async_copy(
                res_v.at[slot, pl.ds(b * PR, PR)],
                out_hbm.at[b, pl.ds(s0, PR)], out_sem.at[slot])

    def wait_out(g, slot):
        s0 = s0w + g * PR
        for b in range(B):
            pltpu.make_async_copy(
                res_v.at[slot, pl.ds(b * PR, PR)],
                out_hbm.at[b, pl.ds(s0, PR)], out_sem.at[slot]).wait()

    def compute(slot):
        for h in range(C // HR):
            was = tuple(w_v[slot, h * HR + i, 0, :] for i in range(HR))
            wbs = tuple(w_v[slot, h * HR + i, 1, :] for i in range(HR))

            def jbody(j, carry):
                was_, wbs_ = carry
                off = j * L
                a1 = segd_v[pl.ds(off, L)]
                a2 = segd_v[pl.ds(D + off, L)]
                for i in range(HR):
                    row = h * HR + i
                    orow = (row % B) * PR + row // B
                    t = tok_v[slot, row, pl.ds(off, L)]
                    p = pe_v[slot, row // B, pl.ds(off, L)]
                    res_v[slot, orow, pl.ds(off, L)] = (
                        t + p + was_[i] * a1 + wbs_[i] * a2)
                return was_, wbs_

            plsc.parallel_loop(0, NV, 1, unroll=2, carry=(was, wbs))(jbody)

    start_in(0)
    start_in(1)
    cd_segd.wait()

    def gbody(g, carry):
        slot = lax.rem(g, 2)
        wait_in(g, slot)

        @pl.when(g >= 2)
        def _():
            wait_out(g - 2, slot)

        compute(slot)
        start_out(g, slot)

        @pl.when(g + 2 < NCH)
        def _():
            start_in(g + 2)

        return carry

    lax.fori_loop(0, NCH, gbody, 0)
    wait_out(NCH - 2, 0)
    wait_out(NCH - 1, 1)


_sc_call = functools.partial(
    pl.kernel,
    out_type=jax.ShapeDtypeStruct((B, S, D), jnp.float32),
    mesh=plsc.VectorSubcoreMesh(core_axis_name="c", subcore_axis_name="s"),
    scratch_types=[
        pltpu.VMEM((NCH, C), jnp.int32),
        pltpu.VMEM((2, C, 2, L), jnp.float32),
        pltpu.VMEM((2 * D,), jnp.float32),
        pltpu.VMEM((2, PR, D), jnp.float32),
        pltpu.VMEM((2, C, D), jnp.float32),
        pltpu.VMEM((2, C, D), jnp.float32),
        pltpu.SemaphoreType.DMA((2,)),
        pltpu.SemaphoreType.DMA((2,)),
        pltpu.SemaphoreType.DMA((2,)),
        pltpu.SemaphoreType.DMA((2,)),
        pltpu.SemaphoreType.DMA,
    ],
)(_body)


def kernel(sequence, segment_labels, tok_table, seg_table, pe):
    seq_sm = sequence.T.reshape(NW, NCH, C).astype(jnp.int32)
    lbl_sm = segment_labels.T.reshape(NW, TPW).astype(jnp.int32)
    w = jnp.broadcast_to(
        jnp.stack([(lbl_sm >= 1), (lbl_sm >= 2)], axis=-1)
        .astype(jnp.float32)[..., None],
        (NW, TPW, 2, L)).reshape(NW, NCH, C, 2, L)
    segd = jnp.concatenate(
        [seg_table[1] - seg_table[0], seg_table[2] - seg_table[1]])
    pe5 = (pe.reshape(S, D) + seg_table[0]).reshape(NW, NCH, PR, D)
    return _sc_call(seq_sm, w, segd, pe5, tok_table)


# speedup vs baseline: 1.1478x; 1.0822x over previous
import functools

import jax
import jax.numpy as jnp
from jax import lax
from jax.experimental import pallas as pl
from jax.experimental.pallas import tpu as pltpu
from jax.experimental.pallas import tpu_sc as plsc

NC, NS, L = 2, 16, 16
NW = NC * NS
B, S, V, D = 4, 2048, 100000, 768
N = B * S
TPW = N // NW
C = 32
NCH = TPW // C
PR = C // B
NV = D // L
HR = 4
PEW = PR * D
AUX = PEW + C * 2 * L


def _body(idx_hbm, oidx_hbm, aux_hbm, segd_hbm, tok_hbm,
          out_hbm, idx_v, oidx_v, aux_v, segd_v, tok_v, res_v,
          tok_sem, aux_sem, out_sem, misc_sem):
    wid = lax.axis_index("s") * NC + lax.axis_index("c")

    cd_idx = pltpu.async_copy(idx_hbm.at[wid], idx_v, misc_sem)
    cd_oidx = pltpu.async_copy(oidx_hbm.at[wid], oidx_v, misc_sem)
    cd_segd = pltpu.async_copy(segd_hbm, segd_v, misc_sem)
    cd_idx.wait()

    def start_in(g):
        slot = lax.rem(g, 2)
        pltpu.async_copy(
            tok_hbm.at[idx_v.at[g]], tok_v.at[slot], tok_sem.at[slot])
        pltpu.async_copy(
            aux_hbm.at[wid, g], aux_v.at[slot], aux_sem.at[slot])

    def wait_in(g, slot):
        pltpu.make_async_copy(
            tok_hbm.at[idx_v.at[g]], tok_v.at[slot], tok_sem.at[slot]).wait()
        pltpu.make_async_copy(
            aux_hbm.at[wid, g], aux_v.at[slot], aux_sem.at[slot]).wait()

    def start_out(g, slot):
        pltpu.async_copy(
            res_v.at[slot], out_hbm.at[oidx_v.at[g]], out_sem.at[slot])

    def wait_out(g, slot):
        pltpu.make_async_copy(
            res_v.at[slot], out_hbm.at[oidx_v.at[g]], out_sem.at[slot]).wait()

    def compute(slot):
        for h in range(C // HR):
            r0 = h * HR
            was = tuple(
                aux_v[slot, pl.ds(PEW + (r0 + i) * 2 * L, L)]
                for i in range(HR))
            wbs = tuple(
                aux_v[slot, pl.ds(PEW + (r0 + i) * 2 * L + L, L)]
                for i in range(HR))

            def jbody(j, carry):
                was_, wbs_ = carry
                off = j * L
                a1 = segd_v[pl.ds(off, L)]
                a2 = segd_v[pl.ds(D + off, L)]
                for i in range(HR):
                    row = r0 + i
                    t = tok_v[slot, row, pl.ds(off, L)]
                    p = aux_v[slot, pl.ds((row // B) * D + off, L)]
                    res_v[slot, row, pl.ds(off, L)] = (
                        t + p + was_[i] * a1 + wbs_[i] * a2)
                return was_, wbs_

            plsc.parallel_loop(0, NV, 1, unroll=2, carry=(was, wbs))(jbody)

    start_in(0)
    start_in(1)
    cd_oidx.wait()
    cd_segd.wait()

    def gbody(g, carry):
        slot = lax.rem(g, 2)
        wait_in(g, slot)

        @pl.when(g >= 2)
        def _():
            wait_out(g - 2, slot)

        compute(slot)
        start_out(g, slot)

        @pl.when(g + 2 < NCH)
        def _():
            start_in(g + 2)

        return carry

    lax.fori_loop(0, NCH, gbody, 0)
    wait_out(NCH - 2, 0)
    wait_out(NCH - 1, 1)


_sc_call = functools.partial(
    pl.kernel,
    out_type=jax.ShapeDtypeStruct((N, D), jnp.float32),
    mesh=plsc.VectorSubcoreMesh(core_axis_name="c", subcore_axis_name="s"),
    scratch_types=[
        pltpu.VMEM((NCH, C), jnp.int32),
        pltpu.VMEM((NCH, C), jnp.int32),
        pltpu.VMEM((2, AUX), jnp.float32),
        pltpu.VMEM((2 * D,), jnp.float32),
        pltpu.VMEM((2, C, D), jnp.float32),
        pltpu.VMEM((2, C, D), jnp.float32),
        pltpu.SemaphoreType.DMA((2,)),
        pltpu.SemaphoreType.DMA((2,)),
        pltpu.SemaphoreType.DMA((2,)),
        pltpu.SemaphoreType.DMA,
    ],
)(_body)


def kernel(sequence, segment_labels, tok_table, seg_table, pe):
    seq_sm = sequence.T.reshape(NW, NCH, C).astype(jnp.int32)
    lbl_sm = segment_labels.T.reshape(NW, TPW).astype(jnp.int32)
    w = jnp.broadcast_to(
        jnp.stack([(lbl_sm >= 1), (lbl_sm >= 2)], axis=-1)
        .astype(jnp.float32)[..., None],
        (NW, TPW, 2, L)).reshape(NW, NCH, C * 2 * L)
    tp = jnp.arange(N, dtype=jnp.int32)
    oidx = ((tp % B) * S + tp // B).reshape(NW, NCH, C)
    segd = jnp.concatenate(
        [seg_table[1] - seg_table[0], seg_table[2] - seg_table[1]])
    pe5 = (pe.reshape(S, D) + seg_table[0]).reshape(NW, NCH, PEW)
    aux = jnp.concatenate([pe5, w], axis=-1)
    out = _sc_call(seq_sm, oidx, aux, segd, tok_table)
    return out.reshape(B, S, D)


# speedup vs baseline: 1.1480x; 1.0002x over previous
import functools

import jax
import jax.numpy as jnp
from jax import lax
from jax.experimental import pallas as pl
from jax.experimental.pallas import tpu as pltpu
from jax.experimental.pallas import tpu_sc as plsc

NC, NS, L = 2, 16, 16
NW = NC * NS
B, S, V, D = 4, 2048, 100000, 768
N = B * S
TPW = N // NW
C = 32
NCH = TPW // C
PR = C // B
NV = D // L
HR = 4
PEW = PR * D
AUX = PEW + C * 2 * L


def _body(idx_hbm, oidx_hbm, aux_hbm, segd_hbm, tok_hbm,
          out_hbm, idx_v, oidx_v, aux_v, segd_v, tok_v, res_v,
          tok_sem, aux_sem, out_sem, misc_sem):
    wid = lax.axis_index("s") * NC + lax.axis_index("c")

    cd_idx = pltpu.async_copy(idx_hbm.at[wid], idx_v, misc_sem)
    cd_oidx = pltpu.async_copy(oidx_hbm.at[wid], oidx_v, misc_sem)
    cd_segd = pltpu.async_copy(segd_hbm, segd_v, misc_sem)
    cd_idx.wait()

    def start_in(g):
        slot = lax.rem(g, 2)
        pltpu.async_copy(
            tok_hbm.at[idx_v.at[g]], tok_v.at[slot], tok_sem.at[slot])
        pltpu.async_copy(
            aux_hbm.at[wid, g], aux_v.at[slot], aux_sem.at[slot])

    def wait_in(g, slot):
        pltpu.make_async_copy(
            tok_hbm.at[idx_v.at[g]], tok_v.at[slot], tok_sem.at[slot]).wait()
        pltpu.make_async_copy(
            aux_hbm.at[wid, g], aux_v.at[slot], aux_sem.at[slot]).wait()

    def start_out(g, slot):
        pltpu.async_copy(
            res_v.at[slot], out_hbm.at[oidx_v.at[g]], out_sem.at[slot])

    def wait_out(g, slot):
        pltpu.make_async_copy(
            res_v.at[slot], out_hbm.at[oidx_v.at[g]], out_sem.at[slot]).wait()

    def compute(slot):
        for h in range(C // HR):
            r0 = h * HR
            was = tuple(
                aux_v[slot, pl.ds(PEW + (r0 + i) * 2 * L, L)]
                for i in range(HR))
            wbs = tuple(
                aux_v[slot, pl.ds(PEW + (r0 + i) * 2 * L + L, L)]
                for i in range(HR))

            def jbody(j, carry):
                was_, wbs_ = carry
                off = j * L
                a1 = segd_v[pl.ds(off, L)]
                a2 = segd_v[pl.ds(D + off, L)]
                for i in range(HR):
                    row = r0 + i
                    t = tok_v[slot, row, pl.ds(off, L)]
                    p = aux_v[slot, pl.ds((row // B) * D + off, L)]
                    res_v[slot, row, pl.ds(off, L)] = (
                        t + p + was_[i] * a1 + wbs_[i] * a2)
                return was_, wbs_

            plsc.parallel_loop(0, NV, 1, unroll=3, carry=(was, wbs))(jbody)

    start_in(0)
    start_in(1)
    cd_oidx.wait()
    cd_segd.wait()

    def gbody(g, carry):
        slot = lax.rem(g, 2)
        wait_in(g, slot)

        @pl.when(g >= 2)
        def _():
            wait_out(g - 2, slot)

        compute(slot)
        start_out(g, slot)

        @pl.when(g + 2 < NCH)
        def _():
            start_in(g + 2)

        return carry

    lax.fori_loop(0, NCH, gbody, 0)
    wait_out(NCH - 2, 0)
    wait_out(NCH - 1, 1)


_sc_call = functools.partial(
    pl.kernel,
    out_type=jax.ShapeDtypeStruct((N, D), jnp.float32),
    mesh=plsc.VectorSubcoreMesh(core_axis_name="c", subcore_axis_name="s"),
    scratch_types=[
        pltpu.VMEM((NCH, C), jnp.int32),
        pltpu.VMEM((NCH, C), jnp.int32),
        pltpu.VMEM((2, AUX), jnp.float32),
        pltpu.VMEM((2 * D,), jnp.float32),
        pltpu.VMEM((2, C, D), jnp.float32),
        pltpu.VMEM((2, C, D), jnp.float32),
        pltpu.SemaphoreType.DMA((2,)),
        pltpu.SemaphoreType.DMA((2,)),
        pltpu.SemaphoreType.DMA((2,)),
        pltpu.SemaphoreType.DMA,
    ],
)(_body)


def kernel(sequence, segment_labels, tok_table, seg_table, pe):
    seq_sm = sequence.T.reshape(NW, NCH, C).astype(jnp.int32)
    lbl_sm = segment_labels.T.reshape(NW, TPW).astype(jnp.int32)
    w = jnp.broadcast_to(
        jnp.stack([(lbl_sm >= 1), (lbl_sm >= 2)], axis=-1)
        .astype(jnp.float32)[..., None],
        (NW, TPW, 2, L)).reshape(NW, NCH, C * 2 * L)
    tp = jnp.arange(N, dtype=jnp.int32)
    oidx = ((tp % B) * S + tp // B).reshape(NW, NCH, C)
    segd = jnp.concatenate(
        [seg_table[1] - seg_table[0], seg_table[2] - seg_table[1]])
    pe5 = (pe.reshape(S, D) + seg_table[0]).reshape(NW, NCH, PEW)
    aux = jnp.concatenate([pe5, w], axis=-1)
    out = _sc_call(seq_sm, oidx, aux, segd, tok_table)
    return out.reshape(B, S, D)
